# baseline (speedup 1.0000x reference)
import functools

import jax
import jax.numpy as jnp
from jax import lax
from jax.experimental import pallas as pl
from jax.experimental.pallas import tpu as pltpu
from jax.experimental.pallas import tpu_sc as plsc

L = 3
D = 128
H = 128
G = 128
C = 10

BLK = 2000


def _relu(x):
    return jnp.maximum(x, 0.0)


def _dot(a, b):
    return jax.lax.dot_general(a, b, (((1,), (0,)), ((), ())),
                               preferred_element_type=jnp.float32)


def _mlp_body(up_ref, b_ref, wu1, bu1, wu2, bu2, wb1, bb1, wb2, bb2,
              wc_u, wc_b, bc, out_ref):
    up = up_ref[...]
    bb = b_ref[...]
    hu = _relu(_dot(up, wu1[...]) + bu1[...])
    hu = _relu(_dot(hu, wu2[...]) + bu2[...])
    hb = _relu(_dot(bb, wb1[...]) + bb1[...])
    hb = _relu(_dot(hb, wb2[...]) + bb2[...])
    out_ref[...] = _relu(_dot(hu, wc_u[...]) + _dot(hb, wc_b[...]) + bc[...])


def _tc_mlp(out_up, out_b, p):
    n = out_up.shape[0]
    grid = n // BLK
    row_spec = pl.BlockSpec((BLK, H), lambda i: (i, 0))
    w_spec = pl.BlockSpec((H, H), lambda i: (0, 0))
    b_spec = pl.BlockSpec((1, H), lambda i: (0, 0))
    return pl.pallas_call(
        _mlp_body,
        grid=(grid,),
        in_specs=[row_spec, row_spec,
                  w_spec, b_spec, w_spec, b_spec,
                  w_spec, b_spec, w_spec, b_spec,
                  w_spec, w_spec, b_spec],
        out_specs=row_spec,
        out_shape=jax.ShapeDtypeStruct((n, H), jnp.float32),
    )(out_up, out_b,
      p["Wu1"], p["bu1"].reshape(1, H), p["Wu2"], p["bu2"].reshape(1, H),
      p["Wb1"], p["bb1"].reshape(1, H), p["Wb2"], p["bb2"].reshape(1, H),
      p["Wc"][:H], p["Wc"][H:], p["bc"].reshape(1, H))


def _pool_body(batch_ref, x_ref, out_ref, acc):
    i = pl.program_id(0)

    @pl.when(i == 0)
    def _():
        acc[...] = jnp.zeros_like(acc)

    b = batch_ref[0, 0, :]
    oh = (b[:, None] == jax.lax.broadcasted_iota(jnp.int32, (BLK, G), 1))
    oh = oh.astype(jnp.float32)
    acc[...] += jax.lax.dot_general(oh, x_ref[...], (((0,), (0,)), ((), ())),
                                    preferred_element_type=jnp.float32)

    @pl.when(i == pl.num_programs(0) - 1)
    def _():
        out_ref[...] = acc[...]


def _tc_pool(x, batch):
    n = x.shape[0]
    grid = n // BLK
    batch3 = batch.astype(jnp.int32).reshape(grid, 1, BLK)
    return pl.pallas_call(
        _pool_body,
        grid=(grid,),
        in_specs=[pl.BlockSpec((1, 1, BLK), lambda i: (i, 0, 0)),
                  pl.BlockSpec((BLK, H), lambda i: (i, 0))],
        out_specs=pl.BlockSpec((G, H), lambda i: (0, 0)),
        out_shape=jax.ShapeDtypeStruct((G, H), jnp.float32),
        scratch_shapes=[pltpu.VMEM((G, H), jnp.float32)],
    )(batch3, x)


def _readout_body(p0, p1, p2, w0, b0, w1, b1, w2, b2, w2f, b2f, out_ref):
    h = _relu(_dot(p0[...], w0[...]) + b0[...])
    h += _relu(_dot(p1[...], w1[...]) + b1[...])
    h += _relu(_dot(p2[...], w2[...]) + b2[...])
    out_ref[...] = _dot(h, w2f[...]) + b2f[...]


def _tc_readout(pooled, lin1, lin2w, lin2b):
    w2f = jnp.zeros((2 * H, 128), jnp.float32).at[:, :C].set(lin2w)
    b2f = jnp.zeros((1, 128), jnp.float32).at[0, :C].set(lin2b)
    full = lambda shape: pl.BlockSpec(shape, lambda: tuple(0 for _ in shape))
    out = pl.pallas_call(
        _readout_body,
        in_specs=[full((G, H))] * 3
        + [full((H, 2 * H)), full((1, 2 * H))] * 3
        + [full((2 * H, 128)), full((1, 128))],
        out_specs=full((G, 128)),
        out_shape=jax.ShapeDtypeStruct((G, 128), jnp.float32),
    )(pooled[0], pooled[1], pooled[2],
      lin1[0]["W"], lin1[0]["b"].reshape(1, 2 * H),
      lin1[1]["W"], lin1[1]["b"].reshape(1, 2 * H),
      lin1[2]["W"], lin1[2]["b"].reshape(1, 2 * H),
      w2f, b2f)
    return out[:, :C]


_CH = 2048


def _sc_segsum_call(nd, ns, epad, rps, npass):
    nblk = 2 * npass
    nrows = epad // 128
    seg = (rps // 16) & ~7
    rem = rps - 15 * seg
    mesh = plsc.VectorSubcoreMesh(core_axis_name="c", subcore_axis_name="s",
                                  num_cores=2, num_subcores=16)

    @functools.partial(
        pl.kernel,
        out_type=jax.ShapeDtypeStruct((nd, D), jnp.float32),
        mesh=mesh,
        scratch_types=[
            pltpu.VMEM_SHARED((rps + 8, D), jnp.float32),
            pltpu.VMEM((nblk + 1, 16), jnp.int32),
            pltpu.VMEM((16, 128), jnp.int32),
            pltpu.VMEM((16, 128), jnp.int32),
            pltpu.VMEM((1, 16), jnp.int32),
            pltpu.VMEM((1, 128), jnp.int32),
            pltpu.VMEM((1, 128), jnp.int32),
            pltpu.VMEM((128, D), jnp.float32),
            pltpu.VMEM((16,), jnp.int32),
            pltpu.VMEM((16,), jnp.int32),
            pltpu.VMEM((16,), jnp.int32),
        ],
    )
    def k(xsrc_h, srcs_h, dsts_h, bsp_h, xdst_h, out_h,
          accum, bvm, esrc, edst, idx16, idxs, idxd, rows, cla, clb, cur):
        sc = lax.axis_index("c")
        tid = lax.axis_index("s")
        lane = lax.iota(jnp.int32, 16)
        sc_v = jnp.zeros((16,), jnp.int32) + sc
        t_lo = tid * seg
        pltpu.sync_copy(bsp_h, bvm)
        for p in range(npass):
            b0 = 2 * p
            r0 = bvm[b0, :]
            r1 = bvm[b0 + 1, :]
            r2 = bvm[b0 + 2, :]
            lo_v = r0 + (r1 - r0) * sc_v
            hi_v = r1 + (r2 - r1) * sc_v
            cla[pl.ds(0, 16)] = lo_v
            clb[pl.ds(0, 16)] = hi_v
            lo_l = cla[pl.ds(0, 16)]
            hi_l = clb[pl.ds(0, 16)]
            e_lo = lo_l[0]
            e_hi = hi_l[0]
            dstbase_v = sc_v * rps + (b0 * rps)
            base_row = sc * rps + b0 * rps

            @pl.when(tid < 15)
            def _():
                pltpu.sync_copy(
                    xdst_h.at[pl.ds(base_row, rps)].at[pl.ds(t_lo, seg)],
                    accum.at[pl.ds(t_lo, seg)])

            @pl.when(tid == 15)
            def _():
                pltpu.sync_copy(
                    xdst_h.at[pl.ds(base_row, rps)].at[pl.ds(15 * seg, rem)],
                    accum.at[pl.ds(15 * seg, rem)])

            plsc.subcore_barrier()

            base_al = (e_lo >> 11) << 11
            nch = (e_hi - base_al + _CH - 1) >> 11
            jmax = jnp.maximum((nch + 15) >> 4, 0)
            cur[pl.ds(0, 16)] = ((lo_l >> 11) << 4) + tid * 16 + lane

            def chunk_body(j, carry):
                base_s = base_al + (tid + j * 16) * _CH

                @pl.when(base_s < e_hi)
                def _():
                    rowv = cur[pl.ds(0, 16)]
                    idx16[0, pl.ds(0, 16)] = rowv
                    pltpu.sync_copy(srcs_h.at[idx16.at[0]], esrc)
                    pltpu.sync_copy(dsts_h.at[idx16.at[0]], edst)
                    for r in range(16):
                        for kk in range(8):
                            s16 = esrc[r, pl.ds(kk * 16, 16)]
                            d16 = edst[r, pl.ds(kk * 16, 16)]
                            u = d16 - dstbase_v
                            oob = (u >> 31) | ((rps - 1 - u) >> 31)
                            idxd[0, pl.ds(kk * 16, 16)] = (
                                (u & ~oob) | (rps & oob))
                            idxs[0, pl.ds(kk * 16, 16)] = s16 & ~oob
                        pltpu.sync_copy(xsrc_h.at[idxs.at[0]], rows)
                        pltpu.sync_copy(rows, accum.at[idxd.at[0]], add=True)
                    cur[pl.ds(0, 16)] = rowv + 256
                return carry

            lax.fori_loop(0, jmax, chunk_body, jnp.int32(0))
            plsc.subcore_barrier()

            @pl.when(tid < 15)
            def _():
                pltpu.sync_copy(
                    accum.at[pl.ds(t_lo, seg)],
                    out_h.at[pl.ds(base_row, rps)].at[pl.ds(t_lo, seg)])

            @pl.when(tid == 15)
            def _():
                pltpu.sync_copy(
                    accum.at[pl.ds(15 * seg, rem)],
                    out_h.at[pl.ds(base_row, rps)].at[pl.ds(15 * seg, rem)])

            if p != npass - 1:
                plsc.subcore_barrier()

    return k


def _sort_edges(src, dst, nd):
    e = src.shape[0]
    pad = (-e) % _CH
    src = jnp.concatenate([src.astype(jnp.int32),
                           jnp.zeros((pad,), jnp.int32)])
    dst = jnp.concatenate([dst.astype(jnp.int32),
                           jnp.full((pad,), nd, jnp.int32)])
    epad = e + pad
    order = jnp.argsort(dst)
    dsts = dst[order]
    srcs = src[order]
    npass = -(nd // -20000)
    rps = nd // (2 * npass)
    assert 2 * rps * npass == nd, (nd, rps, npass)
    nblk = 2 * npass
    bounds = jnp.searchsorted(dsts, jnp.arange(nblk + 1) * rps).astype(jnp.int32)
    bsp = jnp.broadcast_to(bounds[:, None], (nblk + 1, 16))
    return (srcs.reshape(epad // 128, 128), dsts.reshape(epad // 128, 128),
            bsp, epad, rps, npass)


def _segsum_plus(prep, x_src, x_dst):
    srcs2, dsts2, bsp, epad, rps, npass = prep
    nd = x_dst.shape[0]
    k = _sc_segsum_call(nd, x_src.shape[0], epad, rps, npass)
    return k(x_src, srcs2, dsts2, bsp, x_dst)


def kernel(x0, x1, x2, up_index0, up_index1, boundary_src1, boundary_dst1,
           boundary_src2, boundary_dst2, batch0, batch1, batch2, params):
    n0, n1v, n2v = x0.shape[0], x1.shape[0], x2.shape[0]
    prep_up0 = _sort_edges(up_index0[0], up_index0[1], n0)
    prep_up1 = _sort_edges(up_index1[0], up_index1[1], n1v)
    prep_b1 = _sort_edges(boundary_src1, boundary_dst1, n1v)
    prep_b2 = _sort_edges(boundary_src2, boundary_dst2, n2v)

    xs = [x0, x1, x2]
    for l in range(L):
        dims = params["layers"][l]["dims"]
        out_up0 = _segsum_plus(prep_up0, xs[0], xs[0])
        out_up1 = _segsum_plus(prep_up1, xs[1], xs[1])
        out_b1 = _segsum_plus(prep_b1, xs[0], xs[1])
        out_b2 = _segsum_plus(prep_b2, xs[1], xs[2])
        n0 = _tc_mlp(out_up0, xs[0], dims[0])
        n1 = _tc_mlp(out_up1, out_b1, dims[1])
        n2 = _tc_mlp(xs[2], out_b2, dims[2])
        xs = [n0, n1, n2]

    pooled = [_tc_pool(xs[d], [batch0, batch1, batch2][d]) for d in range(3)]
    return _tc_readout(pooled, params["lin1"], params["lin2W"], params["lin2b"])


# speedup vs baseline: 1.0350x; 1.0350x over previous
import functools

import jax
import jax.numpy as jnp
from jax import lax
from jax.experimental import pallas as pl
from jax.experimental.pallas import tpu as pltpu
from jax.experimental.pallas import tpu_sc as plsc

L = 3
D = 128
H = 128
G = 128
C = 10

BLK = 2000


def _relu(x):
    return jnp.maximum(x, 0.0)


def _dot(a, b):
    return jax.lax.dot_general(a, b, (((1,), (0,)), ((), ())),
                               preferred_element_type=jnp.float32)


def _mlp_body(up_ref, b_ref, wu1, bu1, wu2, bu2, wb1, bb1, wb2, bb2,
              wc_u, wc_b, bc, out_ref):
    up = up_ref[...]
    bb = b_ref[...]
    hu = _relu(_dot(up, wu1[...]) + bu1[...])
    hu = _relu(_dot(hu, wu2[...]) + bu2[...])
    hb = _relu(_dot(bb, wb1[...]) + bb1[...])
    hb = _relu(_dot(hb, wb2[...]) + bb2[...])
    out_ref[...] = _relu(_dot(hu, wc_u[...]) + _dot(hb, wc_b[...]) + bc[...])


def _tc_mlp(out_up, out_b, p):
    n = out_up.shape[0]
    grid = n // BLK
    row_spec = pl.BlockSpec((BLK, H), lambda i: (i, 0))
    w_spec = pl.BlockSpec((H, H), lambda i: (0, 0))
    b_spec = pl.BlockSpec((1, H), lambda i: (0, 0))
    return pl.pallas_call(
        _mlp_body,
        grid=(grid,),
        in_specs=[row_spec, row_spec,
                  w_spec, b_spec, w_spec, b_spec,
                  w_spec, b_spec, w_spec, b_spec,
                  w_spec, w_spec, b_spec],
        out_specs=row_spec,
        out_shape=jax.ShapeDtypeStruct((n, H), jnp.float32),
    )(out_up, out_b,
      p["Wu1"], p["bu1"].reshape(1, H), p["Wu2"], p["bu2"].reshape(1, H),
      p["Wb1"], p["bb1"].reshape(1, H), p["Wb2"], p["bb2"].reshape(1, H),
      p["Wc"][:H], p["Wc"][H:], p["bc"].reshape(1, H))


def _pool_body(batch_ref, x_ref, out_ref, acc):
    i = pl.program_id(0)

    @pl.when(i == 0)
    def _():
        acc[...] = jnp.zeros_like(acc)

    b = batch_ref[0, 0, :]
    oh = (b[:, None] == jax.lax.broadcasted_iota(jnp.int32, (BLK, G), 1))
    oh = oh.astype(jnp.float32)
    acc[...] += jax.lax.dot_general(oh, x_ref[...], (((0,), (0,)), ((), ())),
                                    preferred_element_type=jnp.float32)

    @pl.when(i == pl.num_programs(0) - 1)
    def _():
        out_ref[...] = acc[...]


def _tc_pool(x, batch):
    n = x.shape[0]
    grid = n // BLK
    batch3 = batch.astype(jnp.int32).reshape(grid, 1, BLK)
    return pl.pallas_call(
        _pool_body,
        grid=(grid,),
        in_specs=[pl.BlockSpec((1, 1, BLK), lambda i: (i, 0, 0)),
                  pl.BlockSpec((BLK, H), lambda i: (i, 0))],
        out_specs=pl.BlockSpec((G, H), lambda i: (0, 0)),
        out_shape=jax.ShapeDtypeStruct((G, H), jnp.float32),
        scratch_shapes=[pltpu.VMEM((G, H), jnp.float32)],
    )(batch3, x)


def _readout_body(p0, p1, p2, w0, b0, w1, b1, w2, b2, w2f, b2f, out_ref):
    h = _relu(_dot(p0[...], w0[...]) + b0[...])
    h += _relu(_dot(p1[...], w1[...]) + b1[...])
    h += _relu(_dot(p2[...], w2[...]) + b2[...])
    out_ref[...] = _dot(h, w2f[...]) + b2f[...]


def _tc_readout(pooled, lin1, lin2w, lin2b):
    w2f = jnp.zeros((2 * H, 128), jnp.float32).at[:, :C].set(lin2w)
    b2f = jnp.zeros((1, 128), jnp.float32).at[0, :C].set(lin2b)
    full = lambda shape: pl.BlockSpec(shape, lambda: tuple(0 for _ in shape))
    out = pl.pallas_call(
        _readout_body,
        in_specs=[full((G, H))] * 3
        + [full((H, 2 * H)), full((1, 2 * H))] * 3
        + [full((2 * H, 128)), full((1, 128))],
        out_specs=full((G, 128)),
        out_shape=jax.ShapeDtypeStruct((G, 128), jnp.float32),
    )(pooled[0], pooled[1], pooled[2],
      lin1[0]["W"], lin1[0]["b"].reshape(1, 2 * H),
      lin1[1]["W"], lin1[1]["b"].reshape(1, 2 * H),
      lin1[2]["W"], lin1[2]["b"].reshape(1, 2 * H),
      w2f, b2f)
    return out[:, :C]


_CH = 2048


def _sc_segsum_call(nd, ns, epad, rps, npass):
    nblk = 2 * npass
    nrows = epad // 128
    seg = (rps // 16) & ~7
    rem = rps - 15 * seg
    mesh = plsc.VectorSubcoreMesh(core_axis_name="c", subcore_axis_name="s",
                                  num_cores=2, num_subcores=16)

    @functools.partial(
        pl.kernel,
        out_type=jax.ShapeDtypeStruct((nd, D), jnp.float32),
        mesh=mesh,
        scratch_types=[
            pltpu.VMEM_SHARED((rps + 8, D), jnp.float32),
            pltpu.VMEM((nblk + 1, 16), jnp.int32),
            pltpu.VMEM((16, 128), jnp.int32),
            pltpu.VMEM((16, 128), jnp.int32),
            pltpu.VMEM((1, 16), jnp.int32),
            pltpu.VMEM((2, 128), jnp.int32),
            pltpu.VMEM((2, 128), jnp.int32),
            pltpu.VMEM((2, 128, D), jnp.float32),
            pltpu.VMEM((16,), jnp.int32),
            pltpu.VMEM((16,), jnp.int32),
            pltpu.VMEM((16,), jnp.int32),
            pltpu.SemaphoreType.DMA,
            pltpu.SemaphoreType.DMA,
        ],
    )
    def k(xsrc_h, srcs_h, dsts_h, bsp_h, xdst_h, out_h,
          accum, bvm, esrc, edst, idx16, idxs, idxd, rows, cla, clb, cur,
          gsem, ssem):
        sc = lax.axis_index("c")
        tid = lax.axis_index("s")
        lane = lax.iota(jnp.int32, 16)
        sc_v = jnp.zeros((16,), jnp.int32) + sc
        t_lo = tid * seg
        pltpu.sync_copy(bsp_h, bvm)
        for p in range(npass):
            b0 = 2 * p
            r0 = bvm[b0, :]
            r1 = bvm[b0 + 1, :]
            r2 = bvm[b0 + 2, :]
            lo_v = r0 + (r1 - r0) * sc_v
            hi_v = r1 + (r2 - r1) * sc_v
            cla[pl.ds(0, 16)] = lo_v
            clb[pl.ds(0, 16)] = hi_v
            lo_l = cla[pl.ds(0, 16)]
            hi_l = clb[pl.ds(0, 16)]
            e_lo = lo_l[0]
            e_hi = hi_l[0]
            dstbase_v = sc_v * rps + (b0 * rps)
            base_row = sc * rps + b0 * rps

            @pl.when(tid < 15)
            def _():
                pltpu.sync_copy(
                    xdst_h.at[pl.ds(base_row, rps)].at[pl.ds(t_lo, seg)],
                    accum.at[pl.ds(t_lo, seg)])

            @pl.when(tid == 15)
            def _():
                pltpu.sync_copy(
                    xdst_h.at[pl.ds(base_row, rps)].at[pl.ds(15 * seg, rem)],
                    accum.at[pl.ds(15 * seg, rem)])

            plsc.subcore_barrier()

            base_al = (e_lo >> 11) << 11
            nch = (e_hi - base_al + _CH - 1) >> 11
            jmax = jnp.maximum((nch + 15) >> 4, 0)
            cur[pl.ds(0, 16)] = ((lo_l >> 11) << 4) + tid * 16 + lane

            def chunk_body(j, carry):
                base_s = base_al + (tid + j * 16) * _CH

                @pl.when(base_s < e_hi)
                def _():
                    rowv = cur[pl.ds(0, 16)]
                    idx16[0, pl.ds(0, 16)] = rowv
                    pltpu.sync_copy(srcs_h.at[idx16.at[0]], esrc)
                    pltpu.sync_copy(dsts_h.at[idx16.at[0]], edst)
                    gd = [None] * 16
                    sd = [None] * 16
                    for r in range(16):
                        s = r % 2
                        if r >= 2:
                            sd[r - 2].wait()
                        for kk in range(8):
                            s16 = esrc[r, pl.ds(kk * 16, 16)]
                            d16 = edst[r, pl.ds(kk * 16, 16)]
                            u = d16 - dstbase_v
                            oob = (u >> 31) | ((rps - 1 - u) >> 31)
                            idxd[s, pl.ds(kk * 16, 16)] = (
                                (u & ~oob) | (rps & oob))
                            idxs[s, pl.ds(kk * 16, 16)] = s16 & ~oob
                        gd[r] = pltpu.make_async_copy(
                            xsrc_h.at[idxs.at[s]], rows.at[s], gsem)
                        gd[r].start()
                        if r >= 1:
                            gd[r - 1].wait()
                            sd[r - 1] = pltpu.make_async_copy(
                                rows.at[(r - 1) % 2],
                                accum.at[idxd.at[(r - 1) % 2]], ssem)
                            sd[r - 1].start(add=True)
                    for r in (15,):
                        gd[r].wait()
                        sd[r] = pltpu.make_async_copy(
                            rows.at[r % 2], accum.at[idxd.at[r % 2]], ssem)
                        sd[r].start(add=True)
                    for r in (14, 15):
                        sd[r].wait()
                    cur[pl.ds(0, 16)] = rowv + 256
                return carry

            lax.fori_loop(0, jmax, chunk_body, jnp.int32(0))
            plsc.subcore_barrier()

            @pl.when(tid < 15)
            def _():
                pltpu.sync_copy(
                    accum.at[pl.ds(t_lo, seg)],
                    out_h.at[pl.ds(base_row, rps)].at[pl.ds(t_lo, seg)])

            @pl.when(tid == 15)
            def _():
                pltpu.sync_copy(
                    accum.at[pl.ds(15 * seg, rem)],
                    out_h.at[pl.ds(base_row, rps)].at[pl.ds(15 * seg, rem)])

            if p != npass - 1:
                plsc.subcore_barrier()

    return k


def _sort_edges(src, dst, nd):
    e = src.shape[0]
    pad = (-e) % _CH
    src = jnp.concatenate([src.astype(jnp.int32),
                           jnp.zeros((pad,), jnp.int32)])
    dst = jnp.concatenate([dst.astype(jnp.int32),
                           jnp.full((pad,), nd, jnp.int32)])
    epad = e + pad
    order = jnp.argsort(dst)
    dsts = dst[order]
    srcs = src[order]
    npass = -(nd // -20000)
    rps = nd // (2 * npass)
    assert 2 * rps * npass == nd, (nd, rps, npass)
    nblk = 2 * npass
    bounds = jnp.searchsorted(dsts, jnp.arange(nblk + 1) * rps).astype(jnp.int32)
    bsp = jnp.broadcast_to(bounds[:, None], (nblk + 1, 16))
    return (srcs.reshape(epad // 128, 128), dsts.reshape(epad // 128, 128),
            bsp, epad, rps, npass)


def _segsum_plus(prep, x_src, x_dst):
    srcs2, dsts2, bsp, epad, rps, npass = prep
    nd = x_dst.shape[0]
    k = _sc_segsum_call(nd, x_src.shape[0], epad, rps, npass)
    return k(x_src, srcs2, dsts2, bsp, x_dst)


def kernel(x0, x1, x2, up_index0, up_index1, boundary_src1, boundary_dst1,
           boundary_src2, boundary_dst2, batch0, batch1, batch2, params):
    n0, n1v, n2v = x0.shape[0], x1.shape[0], x2.shape[0]
    prep_up0 = _sort_edges(up_index0[0], up_index0[1], n0)
    prep_up1 = _sort_edges(up_index1[0], up_index1[1], n1v)
    prep_b1 = _sort_edges(boundary_src1, boundary_dst1, n1v)
    prep_b2 = _sort_edges(boundary_src2, boundary_dst2, n2v)

    xs = [x0, x1, x2]
    for l in range(L):
        dims = params["layers"][l]["dims"]
        out_up0 = _segsum_plus(prep_up0, xs[0], xs[0])
        out_up1 = _segsum_plus(prep_up1, xs[1], xs[1])
        out_b1 = _segsum_plus(prep_b1, xs[0], xs[1])
        out_b2 = _segsum_plus(prep_b2, xs[1], xs[2])
        n0 = _tc_mlp(out_up0, xs[0], dims[0])
        n1 = _tc_mlp(out_up1, out_b1, dims[1])
        n2 = _tc_mlp(xs[2], out_b2, dims[2])
        xs = [n0, n1, n2]

    pooled = [_tc_pool(xs[d], [batch0, batch1, batch2][d]) for d in range(3)]
    return _tc_readout(pooled, params["lin1"], params["lin2W"], params["lin2b"])
